# baseline (speedup 1.0000x reference)
import functools

import jax
import jax.numpy as jnp
from jax import lax
from jax.experimental import pallas as pl
from jax.experimental.pallas import tpu as pltpu
from jax.experimental.pallas import tpu_sc as plsc

_B, _T = 16, 160000
_N = _B * _T
_NC, _NS = 2, 16
_NW = _NC * _NS
_PER = _N // _NW
_K = 8
_CH = _PER // _K

_mesh = plsc.VectorSubcoreMesh(core_axis_name="c", subcore_axis_name="s")


@functools.partial(
    pl.kernel,
    out_type=jax.ShapeDtypeStruct((_N,), jnp.float32),
    mesh=_mesh,
    scratch_types=[
        pltpu.VMEM((_PER,), jnp.float32),
        pltpu.SemaphoreType.DMA((_K,)),
        pltpu.SemaphoreType.DMA((_K,)),
    ],
)
def _sc_copy(x_hbm, out_hbm, buf, insem, outsem):
    wid = lax.axis_index("s") * _NC + lax.axis_index("c")
    base = wid * _PER
    loads = [
        pltpu.async_copy(
            x_hbm.at[pl.ds(base + i * _CH, _CH)],
            buf.at[pl.ds(i * _CH, _CH)],
            insem.at[i],
        )
        for i in range(_K)
    ]
    stores = []
    for i in range(_K):
        loads[i].wait()
        stores.append(
            pltpu.async_copy(
                buf.at[pl.ds(i * _CH, _CH)],
                out_hbm.at[pl.ds(base + i * _CH, _CH)],
                outsem.at[i],
            )
        )
    for s in stores:
        s.wait()


def kernel(x):
    return _sc_copy(x.reshape(_N)).reshape(_B, _T)


# speedup vs baseline: 1.7828x; 1.7828x over previous
import functools

import jax
import jax.numpy as jnp
from jax import lax
from jax.experimental import pallas as pl
from jax.experimental.pallas import tpu as pltpu
from jax.experimental.pallas import tpu_sc as plsc

_B, _T = 16, 160000
_NC, _NS = 2, 16
_NW = _NC * _NS
_PER = _B * _T // _NW

_mesh = plsc.VectorSubcoreMesh(core_axis_name="c", subcore_axis_name="s")


@functools.partial(
    pl.kernel,
    out_type=jax.ShapeDtypeStruct((_B, _T), jnp.float32),
    mesh=_mesh,
    scratch_types=[pltpu.VMEM((_PER,), jnp.float32)],
)
def _sc_copy(x_hbm, out_hbm, buf):
    wid = lax.axis_index("s") * _NC + lax.axis_index("c")
    row = wid // 2
    col = (wid % 2) * _PER
    pltpu.sync_copy(x_hbm.at[row, pl.ds(col, _PER)], buf)
    pltpu.sync_copy(buf, out_hbm.at[row, pl.ds(col, _PER)])


def kernel(x):
    return _sc_copy(x)


# speedup vs baseline: 1.7849x; 1.0012x over previous
import functools

import jax
import jax.numpy as jnp
from jax import lax
from jax.experimental import pallas as pl
from jax.experimental.pallas import tpu as pltpu
from jax.experimental.pallas import tpu_sc as plsc

_B, _T = 16, 160000
_NC, _NS = 2, 16
_NW = _NC * _NS
_PER = _B * _T // _NW

_K = 5
_CH = _PER // _K

_mesh = plsc.VectorSubcoreMesh(core_axis_name="c", subcore_axis_name="s")


@functools.partial(
    pl.kernel,
    out_type=jax.ShapeDtypeStruct((_B, _T), jnp.float32),
    mesh=_mesh,
    scratch_types=[
        pltpu.VMEM((_PER,), jnp.float32),
        pltpu.SemaphoreType.DMA((_K,)),
        pltpu.SemaphoreType.DMA((_K,)),
    ],
)
def _sc_copy(x_hbm, out_hbm, buf, insem, outsem):
    wid = lax.axis_index("s") * _NC + lax.axis_index("c")
    row = wid // 2
    col = (wid % 2) * _PER
    loads = [
        pltpu.async_copy(
            x_hbm.at[row, pl.ds(col + i * _CH, _CH)],
            buf.at[pl.ds(i * _CH, _CH)],
            insem.at[i],
        )
        for i in range(_K)
    ]
    stores = []
    for i in range(_K):
        loads[i].wait()
        stores.append(
            pltpu.async_copy(
                buf.at[pl.ds(i * _CH, _CH)],
                out_hbm.at[row, pl.ds(col + i * _CH, _CH)],
                outsem.at[i],
            )
        )
    for s in stores:
        s.wait()


def kernel(x):
    return _sc_copy(x)
